# baseline (speedup 1.0000x reference)
import functools

import jax
import jax.numpy as jnp
from jax import lax
from jax.experimental import pallas as pl
from jax.experimental.pallas import tpu as pltpu
from jax.experimental.pallas import tpu_sc as plsc

NUM_NODES = 1000000
EMBED_DIM = 64
BATCH = 16384
N_FIELDS = 26

_NW = 32
_RPW = BATCH // _NW
_CROWS = 16
_NCH = _RPW // _CROWS
_NPAIR = _NCH // 2


def _make_kernel():
    mesh = plsc.VectorSubcoreMesh(core_axis_name="c", subcore_axis_name="s")

    @functools.partial(
        pl.kernel,
        mesh=mesh,
        compiler_params=pltpu.CompilerParams(use_tc_tiling_on_sc=False),
        out_type=jax.ShapeDtypeStruct((BATCH, N_FIELDS, EMBED_DIM), jnp.float32),
        scratch_types=[
            pltpu.VMEM((_RPW, N_FIELDS), jnp.int32),
            pltpu.VMEM((2, _CROWS, N_FIELDS, EMBED_DIM), jnp.float32),
            pltpu.SemaphoreType.DMA,
            pltpu.SemaphoreType.DMA,
            pltpu.SemaphoreType.DMA,
            pltpu.SemaphoreType.DMA,
        ],
    )
    def gather_kernel(ids_hbm, table_hbm, out_hbm, idx_v, rows_v,
                      sem_ga, sem_gb, sem_oa, sem_ob):
        wid = lax.axis_index("s") * 2 + lax.axis_index("c")
        base = wid * _RPW
        pltpu.sync_copy(ids_hbm.at[pl.ds(base, _RPW)], idx_v)

        def start_gather(c, half, sem):
            for r in range(_CROWS):
                pltpu.async_copy(
                    table_hbm.at[idx_v.at[c * _CROWS + r]],
                    rows_v.at[half].at[r], sem)

        def wait_gather(half, sem):
            for r in range(_CROWS):
                pltpu.make_async_copy(
                    table_hbm.at[idx_v.at[0]],
                    rows_v.at[half].at[r], sem).wait()

        def start_out(c, half, sem):
            pltpu.async_copy(
                rows_v.at[half],
                out_hbm.at[pl.ds(base + c * _CROWS, _CROWS)], sem)

        def wait_out(c, half, sem):
            pltpu.make_async_copy(
                rows_v.at[half],
                out_hbm.at[pl.ds(base + c * _CROWS, _CROWS)], sem).wait()

        start_gather(0, 0, sem_ga)

        def body(k, carry):
            c0 = 2 * k
            c1 = 2 * k + 1
            wait_gather(0, sem_ga)

            @pl.when(k > 0)
            def _():
                wait_out(c1 - 2, 1, sem_ob)

            start_gather(c1, 1, sem_gb)
            start_out(c0, 0, sem_oa)
            wait_gather(1, sem_gb)

            @pl.when(k < _NPAIR - 1)
            def _():
                wait_out(c0, 0, sem_oa)
                start_gather(c0 + 2, 0, sem_ga)

            start_out(c1, 1, sem_ob)
            return carry

        lax.fori_loop(0, _NPAIR, body, 0)

        wait_out(_NCH - 2, 0, sem_oa)
        wait_out(_NCH - 1, 1, sem_ob)

    return gather_kernel


_gather = _make_kernel()


def kernel(ids, weight):
    return _gather(ids.astype(jnp.int32), weight)
